# baseline (speedup 1.0000x reference)
import functools

import jax
import jax.numpy as jnp
import numpy as np
from jax import lax
from jax.experimental import pallas as pl
from jax.experimental.pallas import tpu as pltpu
from jax.experimental.pallas import tpu_sc as plsc

N_BINS = 256
OUT = 256
F0_MIN = 50.0
F0_MAX = 1100.0

_MEL_MIN = 1127.0 * np.log(1.0 + F0_MIN / 700.0)
_MEL_MAX = 1127.0 * np.log(1.0 + F0_MAX / 700.0)
_MEL_SCALE = (N_BINS - 1) / (_MEL_MAX - _MEL_MIN)

_NC, _NS = 2, 16
_CHUNK = 256


def _prep_body(f0_ref, pe_ref, uv_ref, b_ref, idx_ref, flog_ref, ct_ref):
    af0 = jnp.abs(f0_ref[...])
    mel = 1127.0 * jnp.log1p(af0 / 700.0)
    binsf = (mel - _MEL_MIN) * _MEL_SCALE
    bins = jnp.clip(binsf.astype(jnp.int32), 0, N_BINS - 1)
    uv = (af0 > 10.0).astype(jnp.int32)
    idx_ref[...] = bins * 2 + uv
    flog_ref[...] = jnp.log1p(af0)
    pe = pe_ref[...]
    base = pe + b_ref[...]
    ct_ref[:, 0:1, :] = (base + uv_ref[0:1])[:, None, :]
    ct_ref[:, 1:2, :] = (base + uv_ref[1:2])[:, None, :]


def _sc_body(ct_hbm, idx_hbm, flog_hbm, w_hbm, out_hbm,
             tab_v, w_v, idx_v, flog_v, outbuf_v):
    c = lax.axis_index("c")
    s = lax.axis_index("s")
    col0 = c * 128
    e_slice = out_hbm.shape[0] // _NS
    ebase0 = s * e_slice

    pltpu.sync_copy(ct_hbm.at[:, pl.ds(col0, 128)], tab_v)
    pltpu.sync_copy(w_hbm.at[pl.ds(col0, 128)], w_v)

    w_vecs = [w_v[pl.ds(16 * k, 16)] for k in range(8)]

    nchunks = e_slice // _CHUNK

    def chunk_body(ci, carry):
        ebase = ebase0 + ci * _CHUNK
        pltpu.sync_copy(idx_hbm.at[pl.ds(ebase, _CHUNK)], idx_v)
        pltpu.sync_copy(flog_hbm.at[pl.ds(ebase, _CHUNK)], flog_v)

        def e_body(eb, carry2):
            e0 = eb * 16
            idx16 = idx_v[pl.ds(e0, 16)]
            flog16 = flog_v[pl.ds(e0, 16)]
            for l in range(16):
                row = idx16[l]
                fl = flog16[l]
                for k in range(8):
                    v = tab_v[row, pl.ds(16 * k, 16)]
                    outbuf_v[e0 + l, pl.ds(16 * k, 16)] = v + fl * w_vecs[k]
            return carry2

        lax.fori_loop(0, _CHUNK // 16, e_body, 0)
        pltpu.sync_copy(outbuf_v,
                        out_hbm.at[pl.ds(ebase, _CHUNK), pl.ds(col0, 128)])
        return carry

    lax.fori_loop(0, nchunks, chunk_body, 0)


def kernel(f0, pitch_embed, uv_embed, W, b):
    B, T = f0.shape
    n = B * T
    b_row = b.reshape(1, OUT)

    idx2d, flog2d, ct3 = pl.pallas_call(
        _prep_body,
        grid=(1,),
        in_specs=[
            pl.BlockSpec((B, T), lambda i: (0, 0)),
            pl.BlockSpec((N_BINS, OUT), lambda i: (0, 0)),
            pl.BlockSpec((2, OUT), lambda i: (0, 0)),
            pl.BlockSpec((1, OUT), lambda i: (0, 0)),
        ],
        out_specs=[
            pl.BlockSpec((B, T), lambda i: (0, 0)),
            pl.BlockSpec((B, T), lambda i: (0, 0)),
            pl.BlockSpec((N_BINS, 2, OUT), lambda i: (0, 0, 0)),
        ],
        out_shape=[
            jax.ShapeDtypeStruct((B, T), jnp.int32),
            jax.ShapeDtypeStruct((B, T), jnp.float32),
            jax.ShapeDtypeStruct((N_BINS, 2, OUT), jnp.float32),
        ],
    )(f0, pitch_embed, uv_embed, b_row)

    idx = idx2d.reshape(n)
    flog = flog2d.reshape(n)
    ctable = ct3.reshape(2 * N_BINS, OUT)
    w_flat = W.reshape(OUT)

    mesh = plsc.VectorSubcoreMesh(
        core_axis_name="c", subcore_axis_name="s",
        num_cores=_NC, num_subcores=_NS)

    sc = functools.partial(
        pl.kernel,
        out_type=jax.ShapeDtypeStruct((n, OUT), jnp.float32),
        mesh=mesh,
        scratch_types=[
            pltpu.VMEM((2 * N_BINS, 128), jnp.float32),
            pltpu.VMEM((128,), jnp.float32),
            pltpu.VMEM((_CHUNK,), jnp.int32),
            pltpu.VMEM((_CHUNK,), jnp.float32),
            pltpu.VMEM((_CHUNK, 128), jnp.float32),
        ],
    )(_sc_body)

    out = sc(ctable, idx, flog, w_flat)
    return out.reshape(B, T, OUT)


# speedup vs baseline: 1.2500x; 1.2500x over previous
import functools

import jax
import jax.numpy as jnp
import numpy as np
from jax import lax
from jax.experimental import pallas as pl
from jax.experimental.pallas import tpu as pltpu
from jax.experimental.pallas import tpu_sc as plsc

N_BINS = 256
OUT = 256
F0_MIN = 50.0
F0_MAX = 1100.0

_MEL_MIN = 1127.0 * np.log(1.0 + F0_MIN / 700.0)
_MEL_MAX = 1127.0 * np.log(1.0 + F0_MAX / 700.0)
_MEL_SCALE = (N_BINS - 1) / (_MEL_MAX - _MEL_MIN)

_NC, _NS = 2, 16
_CHUNK = 128


def _prep_body(f0_ref, pe_ref, uv_ref, b_ref, idx_ref, flog_ref, ct_ref):
    af0 = jnp.abs(f0_ref[...])
    mel = 1127.0 * jnp.log1p(af0 / 700.0)
    binsf = (mel - _MEL_MIN) * _MEL_SCALE
    bins = jnp.clip(binsf.astype(jnp.int32), 0, N_BINS - 1)
    uv = (af0 > 10.0).astype(jnp.int32)
    idx_ref[...] = bins * 2 + uv
    flog_ref[...] = jnp.log1p(af0)
    pe = pe_ref[...]
    base = pe + b_ref[...]
    ct_ref[:, 0:1, :] = (base + uv_ref[0:1])[:, None, :]
    ct_ref[:, 1:2, :] = (base + uv_ref[1:2])[:, None, :]


def _sc_body(ct_hbm, idx_hbm, flog_hbm, w_hbm, out_hbm,
             tab_v, w_v, idx_v, flog_v, ob0, ob1, sem0, sem1):
    c = lax.axis_index("c")
    s = lax.axis_index("s")
    col0 = c * 128
    e_slice = out_hbm.shape[0] // _NS
    ebase0 = s * e_slice

    pltpu.sync_copy(ct_hbm.at[:, pl.ds(col0, 128)], tab_v)
    pltpu.sync_copy(w_hbm.at[pl.ds(col0, 128)], w_v)

    w_vecs = [w_v[pl.ds(16 * k, 16)] for k in range(8)]
    outbufs = (ob0, ob1)
    sems = (sem0, sem1)

    nchunks = e_slice // _CHUNK

    def compute_chunk(ci, buf):
        ebase = ebase0 + ci * _CHUNK
        pltpu.sync_copy(idx_hbm.at[pl.ds(ebase, _CHUNK)], idx_v)
        pltpu.sync_copy(flog_hbm.at[pl.ds(ebase, _CHUNK)], flog_v)

        @plsc.parallel_loop(0, _CHUNK // 16)
        def _(eb):
            e0 = eb * 16
            idx16 = idx_v[pl.ds(e0, 16)]
            flog16 = flog_v[pl.ds(e0, 16)]
            for l in range(16):
                row = idx16[l]
                fl = flog16[l]
                for k in range(8):
                    v = tab_v[row, pl.ds(16 * k, 16)]
                    buf[e0 + l, pl.ds(16 * k, 16)] = v + fl * w_vecs[k]

        return ebase

    def loop2(cj, carry):
        for bsel in range(2):
            @pl.when(cj > 0)
            def _():
                pltpu.make_async_copy(
                    outbufs[bsel],
                    out_hbm.at[pl.ds(ebase0, _CHUNK), pl.ds(col0, 128)],
                    sems[bsel]).wait()
            ebase = compute_chunk(cj * 2 + bsel, outbufs[bsel])
            pltpu.async_copy(
                outbufs[bsel],
                out_hbm.at[pl.ds(ebase, _CHUNK), pl.ds(col0, 128)],
                sems[bsel])
        return carry

    lax.fori_loop(0, nchunks // 2, loop2, 0)
    for bsel in range(2):
        pltpu.make_async_copy(
            outbufs[bsel],
            out_hbm.at[pl.ds(ebase0, _CHUNK), pl.ds(col0, 128)],
            sems[bsel]).wait()


def kernel(f0, pitch_embed, uv_embed, W, b):
    B, T = f0.shape
    n = B * T
    b_row = b.reshape(1, OUT)

    idx2d, flog2d, ct3 = pl.pallas_call(
        _prep_body,
        grid=(1,),
        in_specs=[
            pl.BlockSpec((B, T), lambda i: (0, 0)),
            pl.BlockSpec((N_BINS, OUT), lambda i: (0, 0)),
            pl.BlockSpec((2, OUT), lambda i: (0, 0)),
            pl.BlockSpec((1, OUT), lambda i: (0, 0)),
        ],
        out_specs=[
            pl.BlockSpec((B, T), lambda i: (0, 0)),
            pl.BlockSpec((B, T), lambda i: (0, 0)),
            pl.BlockSpec((N_BINS, 2, OUT), lambda i: (0, 0, 0)),
        ],
        out_shape=[
            jax.ShapeDtypeStruct((B, T), jnp.int32),
            jax.ShapeDtypeStruct((B, T), jnp.float32),
            jax.ShapeDtypeStruct((N_BINS, 2, OUT), jnp.float32),
        ],
    )(f0, pitch_embed, uv_embed, b_row)

    idx = idx2d.reshape(n)
    flog = flog2d.reshape(n)
    ctable = ct3.reshape(2 * N_BINS, OUT)
    w_flat = W.reshape(OUT)

    mesh = plsc.VectorSubcoreMesh(
        core_axis_name="c", subcore_axis_name="s",
        num_cores=_NC, num_subcores=_NS)

    sc = functools.partial(
        pl.kernel,
        out_type=jax.ShapeDtypeStruct((n, OUT), jnp.float32),
        mesh=mesh,
        scratch_types=[
            pltpu.VMEM((2 * N_BINS, 128), jnp.float32),
            pltpu.VMEM((128,), jnp.float32),
            pltpu.VMEM((_CHUNK,), jnp.int32),
            pltpu.VMEM((_CHUNK,), jnp.float32),
            pltpu.VMEM((_CHUNK, 128), jnp.float32),
            pltpu.VMEM((_CHUNK, 128), jnp.float32),
            pltpu.SemaphoreType.DMA,
            pltpu.SemaphoreType.DMA,
        ],
    )(_sc_body)

    out = sc(ctable, idx, flog, w_flat)
    return out.reshape(B, T, OUT)
